# baseline (speedup 1.0000x reference)
import functools

import jax
import jax.numpy as jnp
from jax import lax
from jax.experimental import pallas as pl
from jax.experimental.pallas import tpu as pltpu
from jax.experimental.pallas import tpu_sc as plsc

N = 200
NUM_BUCKETS = 128
K = 21
ROW2 = 2 * N
NPAIR = N // 2
B = 1024
TBL = 160


def _ref_bucket(x):
    return jnp.clip(
        jnp.floor(
            jnp.log(jnp.clip(jnp.abs(x).astype(jnp.float32), 1.0, None))
            / 0.69314718056
        ).astype(jnp.int32),
        0,
        NUM_BUCKETS,
    )


def _thresholds():
    ks = jnp.arange(1, K + 1, dtype=jnp.int32)
    deltas = jnp.arange(-32, 33, dtype=jnp.int32)
    cand = (jnp.int32(1) << ks)[:, None] + deltas[None, :]
    cand = jnp.maximum(cand, 1)
    cb = _ref_bucket(cand)
    big = jnp.where(cb >= ks[:, None], cand, jnp.int32(2**31 - 1))
    return jnp.min(big, axis=1)


def _sc_body(nc, b_per_w,
             ts_hbm, ulo_hbm, uhi_hbm, wv_hbm, pw_hbm, out_hbm, pos_hbm,
             ts_a, ts_b, ulo_v, uhi_v, wv_v, pw_v, stag_a, stag_b,
             pos_stag_v, sem, sem_ts):
    wid = lax.axis_index("s") * nc + lax.axis_index("c")
    pltpu.sync_copy(ulo_hbm, ulo_v)
    pltpu.sync_copy(uhi_hbm, uhi_v)
    pltpu.sync_copy(wv_hbm, wv_v)
    pltpu.sync_copy(pw_hbm, pw_v)
    iota = lax.broadcasted_iota(jnp.int32, (16,), 0)
    one_f = jnp.int32(0x3F800000)

    @pl.when(wid < 25)
    def _():
        for pp in range(4):
            p = wid * 4 + pp
            i0 = 2 * p
            for v in range(25):
                off = v * 16
                if off + 16 <= N:
                    idx = iota + (N - 1 - i0 + off)
                elif off >= N:
                    idx = iota + (N - 1 - (i0 + 1) + off - N)
                else:
                    idx = jnp.where(iota < 8, iota + (N - 1 - i0 + off),
                                    iota - 8 + (N - 1 - (i0 + 1)))
                val = plsc.load_gather(pw_v, [idx])
                pos_stag_v[pl.ds(off, 16)] = val
            pltpu.sync_copy(pos_stag_v, pos_hbm.at[pl.ds(p * ROW2, ROW2)])

    def chain(ext, t1):
        d = t1 - ext
        u = plsc.bitcast(d.astype(jnp.float32), jnp.int32)
        u = jnp.bitwise_and(u, jnp.int32(0x7FFFFFFF))
        e = lax.shift_right_logical(u, 23)
        ulo = plsc.load_gather(ulo_v, [e])
        uhi = plsc.load_gather(uhi_v, [e])
        bk = e + jnp.where(u >= uhi, 1, 0) - jnp.where(u < ulo, 1, 0)
        return plsc.load_gather(wv_v, [bk])

    def compute_batch(stag, ts_v):
        @plsc.parallel_loop(0, N // 4, unroll=1)
        def quad_body(q):
            i0 = 4 * q
            t1s = [
                plsc.load_gather(
                    ts_v, [jnp.full((16,), jnp.minimum(i0 + 1 + r, N - 1),
                                    dtype=jnp.int32)])
                for r in range(4)
            ]
            base = q * 800
            gidx = jnp.where(iota < 8, iota + (N - 16), iota - 8)
            ext_s = plsc.load_gather(ts_v, [gidx])
            for v in range(12):
                ext = ts_v[pl.ds(16 * v, 16)]
                for r in range(4):
                    stag[pl.ds(base + 200 * r + 16 * v, 16)] = chain(
                        ext, t1s[r])
            stag[pl.ds(base + 192, 16)] = chain(
                ext_s, jnp.where(iota < 8, t1s[0], t1s[1]))
            stag[pl.ds(base + 592, 16)] = chain(
                ext_s, jnp.where(iota < 8, t1s[2], t1s[3]))
            wd = chain(ext_s, t1s[3])
            plsc.store_scatter(stag, [gidx + (base + 600)], wd,
                               mask=iota < 8)

    pltpu.async_copy(ts_hbm.at[wid * b_per_w], ts_a, sem_ts)

    def batch_body(m, carry):
        b0 = wid * b_per_w + 2 * m
        pltpu.async_copy(ts_hbm.at[b0 + 1], ts_b, sem_ts)
        pltpu.make_async_copy(ts_hbm.at[b0], ts_a, sem_ts).wait()

        @pl.when(m >= 1)
        def _():
            pltpu.make_async_copy(stag_a, out_hbm.at[b0], sem).wait()

        compute_batch(stag_a, ts_a)
        pltpu.async_copy(stag_a, out_hbm.at[b0], sem)
        pltpu.async_copy(ts_hbm.at[jnp.minimum(b0 + 2, B - 1)], ts_a, sem_ts)
        pltpu.make_async_copy(ts_hbm.at[b0], ts_b, sem_ts).wait()

        @pl.when(m >= 1)
        def _():
            pltpu.make_async_copy(stag_b, out_hbm.at[b0], sem).wait()

        compute_batch(stag_b, ts_b)
        pltpu.async_copy(stag_b, out_hbm.at[b0 + 1], sem)
        return carry

    lax.fori_loop(0, b_per_w // 2, batch_body, 0)
    pltpu.make_async_copy(ts_hbm.at[0], ts_a, sem_ts).wait()
    pltpu.make_async_copy(stag_a, out_hbm.at[0], sem).wait()
    pltpu.make_async_copy(stag_b, out_hbm.at[0], sem).wait()


@jax.jit
def kernel(all_timestamps, ts_w, pos_w):
    info = plsc.get_sparse_core_info()
    nc, ns = info.num_cores, info.num_subcores
    nw = nc * ns
    b_per_w = B // nw
    mesh = plsc.VectorSubcoreMesh(core_axis_name="c", subcore_axis_name="s")
    kfn = pl.kernel(
        functools.partial(_sc_body, nc, b_per_w),
        out_type=(
            jax.ShapeDtypeStruct((B, N * N), jnp.float32),
            jax.ShapeDtypeStruct((N * N,), jnp.float32),
        ),
        mesh=mesh,
        compiler_params=pltpu.CompilerParams(needs_layout_passes=False),
        scratch_types=[
            pltpu.VMEM((N,), jnp.int32),
            pltpu.VMEM((N,), jnp.int32),
            pltpu.VMEM((TBL,), jnp.int32),
            pltpu.VMEM((TBL,), jnp.int32),
            pltpu.VMEM((TBL,), jnp.float32),
            pltpu.VMEM((ROW2,), jnp.float32),
            pltpu.VMEM((N * N,), jnp.float32),
            pltpu.VMEM((N * N,), jnp.float32),
            pltpu.VMEM((ROW2,), jnp.float32),
            pltpu.SemaphoreType.DMA,
            pltpu.SemaphoreType.DMA,
        ],
    )

    thr = _thresholds()
    ubits = lax.bitcast_convert_type(thr.astype(jnp.float32), jnp.int32)
    maxfin = jnp.int32(0x7F7FFFFF)
    onef = jnp.int32(0x3F800000)
    ulo = jnp.full((TBL,), onef, jnp.int32).at[128 : 128 + K].set(ubits)
    uhi = jnp.full((TBL,), maxfin, jnp.int32).at[127 : 127 + K].set(ubits)
    ulo = ulo.at[0].set(0)
    wv = jnp.zeros((TBL,), jnp.float32).at[127 : 159].set(ts_w[:32])
    wv = wv.at[0].set(ts_w[0])
    pw = jnp.zeros((ROW2,), jnp.float32).at[: 2 * N - 1].set(pos_w)
    rel_ts, rel_pos = kfn(all_timestamps, ulo, uhi, wv, pw)
    return rel_pos.reshape(1, N, N), rel_ts.reshape(B, N, N)


# speedup vs baseline: 1.0264x; 1.0264x over previous
import functools

import jax
import jax.numpy as jnp
from jax import lax
from jax.experimental import pallas as pl
from jax.experimental.pallas import tpu as pltpu
from jax.experimental.pallas import tpu_sc as plsc

N = 200
NUM_BUCKETS = 128
K = 21
ROW2 = 2 * N
NPAIR = N // 2
B = 1024
TBL = 160


def _ref_bucket(x):
    return jnp.clip(
        jnp.floor(
            jnp.log(jnp.clip(jnp.abs(x).astype(jnp.float32), 1.0, None))
            / 0.69314718056
        ).astype(jnp.int32),
        0,
        NUM_BUCKETS,
    )


def _thresholds():
    ks = jnp.arange(1, K + 1, dtype=jnp.int32)
    deltas = jnp.arange(-32, 33, dtype=jnp.int32)
    cand = (jnp.int32(1) << ks)[:, None] + deltas[None, :]
    cand = jnp.maximum(cand, 1)
    cb = _ref_bucket(cand)
    big = jnp.where(cb >= ks[:, None], cand, jnp.int32(2**31 - 1))
    return jnp.min(big, axis=1)


def _sc_body(nc, b_per_w,
             ts_hbm, ulo_hbm, uhi_hbm, wv_hbm, pw_hbm, out_hbm, pos_hbm,
             ts_a, ts_b, ulo_v, uhi_v, wv_v, pw_v, stag_a, stag_b,
             pos_stag_v, sem, sem_ts):
    wid = lax.axis_index("s") * nc + lax.axis_index("c")
    pltpu.sync_copy(ulo_hbm, ulo_v)
    pltpu.sync_copy(uhi_hbm, uhi_v)
    pltpu.sync_copy(wv_hbm, wv_v)
    pltpu.sync_copy(pw_hbm, pw_v)
    iota = lax.broadcasted_iota(jnp.int32, (16,), 0)
    one_f = jnp.int32(0x3F800000)

    @pl.when(wid < 25)
    def _():
        for pp in range(4):
            p = wid * 4 + pp
            i0 = 2 * p
            for v in range(25):
                off = v * 16
                if off + 16 <= N:
                    idx = iota + (N - 1 - i0 + off)
                elif off >= N:
                    idx = iota + (N - 1 - (i0 + 1) + off - N)
                else:
                    idx = jnp.where(iota < 8, iota + (N - 1 - i0 + off),
                                    iota - 8 + (N - 1 - (i0 + 1)))
                val = plsc.load_gather(pw_v, [idx])
                pos_stag_v[pl.ds(off, 16)] = val
            pltpu.sync_copy(pos_stag_v, pos_hbm.at[pl.ds(p * ROW2, ROW2)])

    def chain(ext, t1):
        d = t1 - ext
        u = plsc.bitcast(d.astype(jnp.float32), jnp.int32)
        u = jnp.bitwise_and(u, jnp.int32(0x7FFFFFFF))
        e = lax.shift_right_logical(u, 23)
        ulo = plsc.load_gather(ulo_v, [e])
        uhi = plsc.load_gather(uhi_v, [e])
        bk = e + jnp.where(u >= uhi, 1, 0) - jnp.where(u < ulo, 1, 0)
        return plsc.load_gather(wv_v, [bk])

    def compute_batch(stag, ts_v):
        @plsc.parallel_loop(0, NPAIR, unroll=2)
        def pair_body(p):
            va = plsc.load_gather(
                ts_v, [jnp.full((16,), 2 * p + 1, dtype=jnp.int32)])
            vb = plsc.load_gather(
                ts_v, [jnp.full((16,), jnp.minimum(2 * p + 2, N - 1),
                                dtype=jnp.int32)])
            base = p * ROW2
            for v in range(25):
                off = v * 16
                if off + 16 <= N:
                    ext = ts_v[pl.ds(off, 16)]
                    t1 = va
                elif off >= N:
                    ext = ts_v[pl.ds(off - N, 16)]
                    t1 = vb
                else:
                    gidx = jnp.where(iota < 8, iota + off, iota - 8)
                    ext = plsc.load_gather(ts_v, [gidx])
                    t1 = jnp.where(iota < 8, va, vb)
                stag[pl.ds(base + off, 16)] = chain(ext, t1)

    pltpu.async_copy(ts_hbm.at[wid * b_per_w], ts_a, sem_ts)

    def batch_body(m, carry):
        b0 = wid * b_per_w + 2 * m
        pltpu.async_copy(ts_hbm.at[b0 + 1], ts_b, sem_ts)
        pltpu.make_async_copy(ts_hbm.at[b0], ts_a, sem_ts).wait()

        @pl.when(m >= 1)
        def _():
            pltpu.make_async_copy(stag_a, out_hbm.at[b0], sem).wait()

        compute_batch(stag_a, ts_a)
        pltpu.async_copy(stag_a, out_hbm.at[b0], sem)
        pltpu.async_copy(ts_hbm.at[jnp.minimum(b0 + 2, B - 1)], ts_a, sem_ts)
        pltpu.make_async_copy(ts_hbm.at[b0], ts_b, sem_ts).wait()

        @pl.when(m >= 1)
        def _():
            pltpu.make_async_copy(stag_b, out_hbm.at[b0], sem).wait()

        compute_batch(stag_b, ts_b)
        pltpu.async_copy(stag_b, out_hbm.at[b0 + 1], sem)
        return carry

    lax.fori_loop(0, b_per_w // 2, batch_body, 0)
    pltpu.make_async_copy(ts_hbm.at[0], ts_a, sem_ts).wait()
    pltpu.make_async_copy(stag_a, out_hbm.at[0], sem).wait()
    pltpu.make_async_copy(stag_b, out_hbm.at[0], sem).wait()


@jax.jit
def kernel(all_timestamps, ts_w, pos_w):
    info = plsc.get_sparse_core_info()
    nc, ns = info.num_cores, info.num_subcores
    nw = nc * ns
    b_per_w = B // nw
    mesh = plsc.VectorSubcoreMesh(core_axis_name="c", subcore_axis_name="s")
    kfn = pl.kernel(
        functools.partial(_sc_body, nc, b_per_w),
        out_type=(
            jax.ShapeDtypeStruct((B, N * N), jnp.float32),
            jax.ShapeDtypeStruct((N * N,), jnp.float32),
        ),
        mesh=mesh,
        compiler_params=pltpu.CompilerParams(needs_layout_passes=False),
        scratch_types=[
            pltpu.VMEM((N,), jnp.int32),
            pltpu.VMEM((N,), jnp.int32),
            pltpu.VMEM((TBL,), jnp.int32),
            pltpu.VMEM((TBL,), jnp.int32),
            pltpu.VMEM((TBL,), jnp.float32),
            pltpu.VMEM((ROW2,), jnp.float32),
            pltpu.VMEM((N * N,), jnp.float32),
            pltpu.VMEM((N * N,), jnp.float32),
            pltpu.VMEM((ROW2,), jnp.float32),
            pltpu.SemaphoreType.DMA,
            pltpu.SemaphoreType.DMA,
        ],
    )

    thr = _thresholds()
    ubits = lax.bitcast_convert_type(thr.astype(jnp.float32), jnp.int32)
    maxfin = jnp.int32(0x7F7FFFFF)
    onef = jnp.int32(0x3F800000)
    ulo = jnp.full((TBL,), onef, jnp.int32).at[128 : 128 + K].set(ubits)
    uhi = jnp.full((TBL,), maxfin, jnp.int32).at[127 : 127 + K].set(ubits)
    ulo = ulo.at[0].set(0)
    wv = jnp.zeros((TBL,), jnp.float32).at[127 : 159].set(ts_w[:32])
    wv = wv.at[0].set(ts_w[0])
    pw = jnp.zeros((ROW2,), jnp.float32).at[: 2 * N - 1].set(pos_w)
    rel_ts, rel_pos = kfn(all_timestamps, ulo, uhi, wv, pw)
    return rel_pos.reshape(1, N, N), rel_ts.reshape(B, N, N)
